# baseline (speedup 1.0000x reference)
import jax
import jax.numpy as jnp
from jax.experimental import pallas as pl
from jax.experimental.pallas import tpu as pltpu

VOCAB = 1000000
H = 64
CTX = 200
BLOCK_V = 25000
NB = VOCAB // BLOCK_V


def _body(ids_ref, idx_ref, numf_ref, tok_hbm, node_ref, parent_ref,
          lang_ref, numWT_ref, hidWT_ref, hidb_ref, swW_ref, swb_ref,
          prefW_ref, prefb_ref, switch_out, pref_out,
          tok_scratch, hid_scratch, sem):
    i = pl.program_id(0)

    @pl.when(i == 0)
    def _prologue():
        def issue(t, _):
            pltpu.make_async_copy(
                tok_hbm.at[pl.ds(ids_ref[t], 1), :],
                tok_scratch.at[pl.ds(t, 1), :],
                sem,
            ).start()
            return 0
        jax.lax.fori_loop(0, CTX, issue, 0)

        def wait(t, _):
            pltpu.make_async_copy(
                tok_hbm.at[pl.ds(ids_ref[t], 1), :],
                tok_scratch.at[pl.ds(t, 1), :],
                sem,
            ).wait()
            return 0
        jax.lax.fori_loop(0, CTX, wait, 0)

        token_summary = jnp.sum(tok_scratch[...], axis=0, keepdims=True) * (1.0 / CTX)
        node_row = node_ref[pl.ds(idx_ref[0], 1), :]
        parent_row = parent_ref[pl.ds(idx_ref[1], 1), :]
        lang_row = lang_ref[pl.ds(idx_ref[2], 1), :]
        num_proj = (numf_ref[0] * numWT_ref[0:1, :]
                    + numf_ref[1] * numWT_ref[1:2, :]
                    + numf_ref[2] * numWT_ref[2:3, :])
        feature_summary = node_row + parent_row + lang_row + num_proj
        concat = jnp.concatenate([token_summary, feature_summary], axis=1)
        hidden = jnp.tanh(
            jax.lax.dot_general(concat, hidWT_ref[...],
                                (((1,), (0,)), ((), ())),
                                preferred_element_type=jnp.float32)
            + hidb_ref[...])
        hid_scratch[...] = hidden
        switch_out[...] = (jnp.sum(swW_ref[...] * hidden, axis=1, keepdims=True)
                           + swb_ref[0])

    logits = jax.lax.dot_general(
        hid_scratch[...], prefW_ref[0],
        (((1,), (1,)), ((), ())),
        preferred_element_type=jnp.float32)
    pref_out[0] = logits + prefb_ref[0]


def kernel(prefix_ids, node_idx, parent_idx, lang_idx, numeric_features,
           token_table, node_table, parent_table, lang_table,
           num_W, num_b, hid_W, hid_b, sw_W, sw_b, pref_W, pref_b):
    ids = prefix_ids[-CTX:].astype(jnp.int32)
    idx3 = jnp.stack([jnp.asarray(node_idx, jnp.int32),
                      jnp.asarray(parent_idx, jnp.int32),
                      jnp.asarray(lang_idx, jnp.int32)])
    pref_W3 = pref_W.reshape(NB, BLOCK_V, H)
    pref_b2 = pref_b.reshape(NB, 1, BLOCK_V)

    smem = pl.BlockSpec(memory_space=pltpu.MemorySpace.SMEM)
    vmem_full = pl.BlockSpec(memory_space=pltpu.MemorySpace.VMEM)

    switch, pref = pl.pallas_call(
        _body,
        grid=(NB,),
        in_specs=[
            smem,
            smem,
            smem,
            pl.BlockSpec(memory_space=pltpu.MemorySpace.HBM),
            vmem_full,
            vmem_full,
            vmem_full,
            vmem_full,
            vmem_full,
            vmem_full,
            vmem_full,
            smem,
            pl.BlockSpec((1, BLOCK_V, H), lambda i: (i, 0, 0)),
            pl.BlockSpec((1, 1, BLOCK_V), lambda i: (i, 0, 0)),
        ],
        out_specs=[
            pl.BlockSpec((1, 1), lambda i: (0, 0)),
            pl.BlockSpec((1, 1, BLOCK_V), lambda i: (i, 0, 0)),
        ],
        out_shape=[
            jax.ShapeDtypeStruct((1, 1), jnp.float32),
            jax.ShapeDtypeStruct((NB, 1, BLOCK_V), jnp.float32),
        ],
        scratch_shapes=[
            pltpu.VMEM((CTX, H), jnp.float32),
            pltpu.VMEM((1, H), jnp.float32),
            pltpu.SemaphoreType.DMA,
        ],
    )(ids, idx3, numeric_features, token_table, node_table, parent_table,
      lang_table, num_W.T, hid_W.T, hid_b.reshape(1, H), sw_W,
      sw_b, pref_W3, pref_b2)

    return (switch[0, 0], pref.reshape(VOCAB))
